# Initial kernel scaffold; baseline (speedup 1.0000x reference)
import jax
import jax.numpy as jnp
from jax.experimental import pallas as pl


def kernel(x, edge_index, W1, b1, W2, b2, Wa1, ba1, Wa2, ba2, Wc1, bc1, Wc2, bc2):
    raise NotImplementedError("write your pallas kernel here")


# speedup vs baseline: 10.4133x; 10.4133x over previous
import functools

import jax
import jax.numpy as jnp
from jax import lax
from jax.experimental import pallas as pl
from jax.experimental.pallas import tpu as pltpu
from jax.experimental.pallas import tpu_sc as plsc

N = 10000
E = 320000
NC = 2
NS = 16
CH = 128
R0 = 632
RLAST = N - (NS - 1) * R0

_MESH = dict(core_axis_name="c", subcore_axis_name="s", num_cores=NC,
             num_subcores=NS)


def _rowwise_copy(s, make_src, make_dst):
    row0 = s * R0

    @pl.when(s < NS - 1)
    def _():
        pltpu.sync_copy(make_src(row0, R0), make_dst(row0, R0))

    @pl.when(s == NS - 1)
    def _():
        pltpu.sync_copy(make_src((NS - 1) * R0, RLAST),
                        make_dst((NS - 1) * R0, RLAST))


def _chunk_range(s, total_chunks):
    base = total_chunks // NS
    rem = total_chunks % NS
    n = base + jnp.where(s < rem, 1, 0)
    start = base * s + jnp.minimum(s, rem)
    return start, n


@functools.partial(
    pl.kernel,
    out_type=jax.ShapeDtypeStruct((NC, N, 8), jnp.float32),
    mesh=plsc.VectorSubcoreMesh(**_MESH),
    scratch_types=[
        pltpu.VMEM((CH,), jnp.int32),
        pltpu.VMEM((CH, 8), jnp.float32),
        pltpu.VMEM_SHARED((N, 8), jnp.float32),
    ],
)
def _sc_deg(dst_hbm, const_hbm, out_hbm, didx, ones_rows, accum):
    c = lax.axis_index("c")
    s = lax.axis_index("s")
    pltpu.sync_copy(const_hbm.at[pl.ds(N, CH)], ones_rows)
    _rowwise_copy(s, lambda r, m: const_hbm.at[pl.ds(r, m)],
                  lambda r, m: accum.at[pl.ds(r, m)])
    plsc.subcore_barrier()

    half = (E // 2) // CH
    start, n = _chunk_range(s, half)

    def body(k, carry):
        off = c * (E // 2) + (start + k) * CH
        pltpu.sync_copy(dst_hbm.at[pl.ds(off, CH)], didx)
        pltpu.sync_copy(ones_rows, accum.at[didx], add=True)
        return carry

    lax.fori_loop(0, n, body, 0)
    plsc.subcore_barrier()
    _rowwise_copy(s, lambda r, m: accum.at[pl.ds(r, m)],
                  lambda r, m: out_hbm.at[c, pl.ds(r, m)])


@functools.partial(
    pl.kernel,
    out_type=jax.ShapeDtypeStruct((NC * N, 128), jnp.float32),
    mesh=plsc.VectorSubcoreMesh(**_MESH),
    scratch_types=[
        pltpu.VMEM((CH,), jnp.int32),
        pltpu.VMEM((CH,), jnp.int32),
        pltpu.VMEM((CH, 128), jnp.float32),
        pltpu.VMEM_SHARED((N, 128), jnp.float32),
        pltpu.SemaphoreType.DMA,
    ],
)
def _sc_agg(y_hbm, srcp_hbm, dst_hbm, out_hbm, gidx, didx, rows, accum, sem):
    c = lax.axis_index("c")
    s = lax.axis_index("s")
    _rowwise_copy(s, lambda r, m: y_hbm.at[pl.ds(c * N + r, m)],
                  lambda r, m: accum.at[pl.ds(r, m)])
    plsc.subcore_barrier()

    start, n = _chunk_range(s, E // CH)

    def body(k, carry):
        off = (start + k) * CH
        pltpu.sync_copy(srcp_hbm.at[pl.ds(c * E + off, CH)], gidx)
        pltpu.sync_copy(dst_hbm.at[pl.ds(off, CH)], didx)
        pltpu.async_copy(y_hbm.at[gidx], rows, sem).wait()
        pltpu.sync_copy(rows, accum.at[didx], add=True)
        return carry

    lax.fori_loop(0, n, body, 0)
    plsc.subcore_barrier()
    _rowwise_copy(s, lambda r, m: accum.at[pl.ds(r, m)],
                  lambda r, m: out_hbm.at[pl.ds(c * N + r, m)])


_BM = 1000
_GRID = N // _BM


def _t1_body(x_ref, w_ref, degp_ref, y_ref, dinv_ref):
    deg = degp_ref[0, :, 0:1] + degp_ref[1, :, 0:1]
    dinv = lax.rsqrt(deg)
    xw = jnp.dot(x_ref[:], w_ref[:], preferred_element_type=jnp.float32)
    y = xw * dinv
    y_ref[0] = y[:, :128]
    y_ref[1] = y[:, 128:]
    dinv_ref[:] = dinv


def _t1(x, w1, degp):
    return pl.pallas_call(
        _t1_body,
        grid=(_GRID,),
        in_specs=[
            pl.BlockSpec((_BM, 128), lambda i: (i, 0)),
            pl.BlockSpec((128, 256), lambda i: (0, 0)),
            pl.BlockSpec((NC, _BM, 8), lambda i: (0, i, 0)),
        ],
        out_specs=[
            pl.BlockSpec((NC, _BM, 128), lambda i: (0, i, 0)),
            pl.BlockSpec((_BM, 1), lambda i: (i, 0)),
        ],
        out_shape=[
            jax.ShapeDtypeStruct((NC, N, 128), jnp.float32),
            jax.ShapeDtypeStruct((N, 1), jnp.float32),
        ],
    )(x, w1, degp)


def _t2_body(a_ref, dinv_ref, b_ref, w_ref, y_ref):
    dinv = dinv_ref[:]
    x1a = jnp.maximum(a_ref[0] * dinv + b_ref[0], 0.0)
    x1b = jnp.maximum(a_ref[1] * dinv + b_ref[1], 0.0)
    h = (jnp.dot(x1a, w_ref[0], preferred_element_type=jnp.float32)
         + jnp.dot(x1b, w_ref[1], preferred_element_type=jnp.float32))
    y = h * dinv
    y_ref[0] = y[:, :128]
    y_ref[1] = y[:, 128:]


def _t2(a1, dinv, b1r, w2r):
    return pl.pallas_call(
        _t2_body,
        grid=(_GRID,),
        in_specs=[
            pl.BlockSpec((NC, _BM, 128), lambda i: (0, i, 0)),
            pl.BlockSpec((_BM, 1), lambda i: (i, 0)),
            pl.BlockSpec((NC, 1, 128), lambda i: (0, 0, 0)),
            pl.BlockSpec((NC, 128, 256), lambda i: (0, 0, 0)),
        ],
        out_specs=pl.BlockSpec((NC, _BM, 128), lambda i: (0, i, 0)),
        out_shape=jax.ShapeDtypeStruct((NC, N, 128), jnp.float32),
    )(a1, dinv, b1r, w2r)


def _t3_body(a_ref, dinv_ref, b2_ref, wa1_ref, ba1_ref, wa2_ref, ba2_ref,
             wc1_ref, bc1_ref, wc2_ref, bc2_ref, lg_ref, val_ref, pr_ref):
    dinv = dinv_ref[:]
    x2a = jnp.maximum(a_ref[0] * dinv + b2_ref[0], 0.0)
    x2b = jnp.maximum(a_ref[1] * dinv + b2_ref[1], 0.0)
    ha = jnp.maximum(
        jnp.dot(x2a, wa1_ref[0], preferred_element_type=jnp.float32)
        + jnp.dot(x2b, wa1_ref[1], preferred_element_type=jnp.float32)
        + ba1_ref[:], 0.0)
    logits = jnp.dot(ha, wa2_ref[:], preferred_element_type=jnp.float32) + ba2_ref[:]
    hc = jnp.maximum(
        jnp.dot(x2a, wc1_ref[0], preferred_element_type=jnp.float32)
        + jnp.dot(x2b, wc1_ref[1], preferred_element_type=jnp.float32)
        + bc1_ref[:], 0.0)
    val = jnp.dot(hc, wc2_ref[:], preferred_element_type=jnp.float32) + bc2_ref[:]
    m = jnp.max(logits, axis=-1, keepdims=True)
    ex = jnp.exp(logits - m)
    lg_ref[:] = logits
    val_ref[:] = val
    pr_ref[:] = ex / jnp.sum(ex, axis=-1, keepdims=True)


def _t3(a2, dinv, b2r, wa1r, ba1, wa2, ba2, wc1r, bc1, wc2, bc2):
    return pl.pallas_call(
        _t3_body,
        grid=(_GRID,),
        in_specs=[
            pl.BlockSpec((NC, _BM, 128), lambda i: (0, i, 0)),
            pl.BlockSpec((_BM, 1), lambda i: (i, 0)),
            pl.BlockSpec((NC, 1, 128), lambda i: (0, 0, 0)),
            pl.BlockSpec((NC, 128, 128), lambda i: (0, 0, 0)),
            pl.BlockSpec((1, 128), lambda i: (0, 0)),
            pl.BlockSpec((128, 8), lambda i: (0, 0)),
            pl.BlockSpec((1, 8), lambda i: (0, 0)),
            pl.BlockSpec((NC, 128, 128), lambda i: (0, 0, 0)),
            pl.BlockSpec((1, 128), lambda i: (0, 0)),
            pl.BlockSpec((128, 1), lambda i: (0, 0)),
            pl.BlockSpec((1, 1), lambda i: (0, 0)),
        ],
        out_specs=[
            pl.BlockSpec((_BM, 8), lambda i: (i, 0)),
            pl.BlockSpec((_BM, 1), lambda i: (i, 0)),
            pl.BlockSpec((_BM, 8), lambda i: (i, 0)),
        ],
        out_shape=[
            jax.ShapeDtypeStruct((N, 8), jnp.float32),
            jax.ShapeDtypeStruct((N, 1), jnp.float32),
            jax.ShapeDtypeStruct((N, 8), jnp.float32),
        ],
    )(a2, dinv, b2r, wa1r, ba1, wa2, ba2, wc1r, bc1, wc2, bc2)


def kernel(x, edge_index, W1, b1, W2, b2, Wa1, ba1, Wa2, ba2, Wc1, bc1,
           Wc2, bc2):
    src = edge_index[0]
    dst = edge_index[1]
    srcp = jnp.concatenate([src, src + N])
    const = jnp.concatenate([jnp.full((N, 8), 0.5, jnp.float32),
                             jnp.ones((CH, 8), jnp.float32)])

    degp = _sc_deg(dst, const)
    y1, dinv = _t1(x, W1, degp)

    a1 = _sc_agg(y1.reshape(NC * N, 128), srcp, dst).reshape(NC, N, 128)
    b1r = b1.reshape(NC, 1, 128)
    w2r = W2.reshape(NC, 128, 256)
    y2 = _t2(a1, dinv, b1r, w2r)

    a2 = _sc_agg(y2.reshape(NC * N, 128), srcp, dst).reshape(NC, N, 128)
    logits, values, probs = _t3(
        a2, dinv, b2.reshape(NC, 1, 128),
        Wa1.reshape(NC, 128, 128), ba1.reshape(1, 128), Wa2,
        ba2.reshape(1, 8),
        Wc1.reshape(NC, 128, 128), bc1.reshape(1, 128), Wc2,
        bc2.reshape(1, 1))
    return (logits, values, probs)
